# Initial kernel scaffold; baseline (speedup 1.0000x reference)
import jax
import jax.numpy as jnp
from jax.experimental import pallas as pl


def kernel(inputs, W_head, b_head, g_head, be_head, W_blocks, b_blocks, g_blocks, be_blocks):
    raise NotImplementedError("write your pallas kernel here")


# speedup vs baseline: 1.0009x; 1.0009x over previous
import functools

import jax
import jax.numpy as jnp
from jax.experimental import pallas as pl
from jax.experimental.pallas import tpu as pltpu

KK = 20
NBLK = 7
NFILT = 16
CIN = 4
BB = 4
NN = 4096

ROW_BLK = 512


def _dist_body(xt_blk_ref, xt_all_ref, sq_blk_ref, sq_all_ref, out_ref):
    a = xt_blk_ref[0]
    x = xt_all_ref[0]
    m = jax.lax.dot_general(a, x, (((0,), (0,)), ((), ())),
                            preferred_element_type=jnp.float32)
    x_inner = -2.0 * m
    sq_r = sq_blk_ref[0]
    sq_a = sq_all_ref[0]
    d = (jnp.transpose(sq_r) + x_inner) + sq_a
    out_ref[0] = -d


def _neg_adj_pallas(xt):
    b, n, c = xt.shape
    xt_t = jnp.swapaxes(xt, 1, 2)
    sq = jnp.sum(xt * xt, axis=-1)[:, None, :]
    grid = (b, n // ROW_BLK)
    return pl.pallas_call(
        _dist_body,
        grid=grid,
        in_specs=[
            pl.BlockSpec((1, c, ROW_BLK), lambda i, j: (i, 0, j)),
            pl.BlockSpec((1, c, n), lambda i, j: (i, 0, 0)),
            pl.BlockSpec((1, 1, ROW_BLK), lambda i, j: (i, 0, j)),
            pl.BlockSpec((1, 1, n), lambda i, j: (i, 0, 0)),
        ],
        out_specs=pl.BlockSpec((1, ROW_BLK, n), lambda i, j: (i, j, 0)),
        out_shape=jax.ShapeDtypeStruct((b, n, n), jnp.float32),
    )(xt_t, xt_t, sq, sq)


def _knn_graph(x, k, dilation):
    xt = jnp.squeeze(x, -1).transpose(0, 2, 1)
    b, n, _ = xt.shape
    neg_adj = _neg_adj_pallas(xt)
    _, nn_idx = jax.lax.top_k(neg_adj, k * dilation)
    nn_idx = nn_idx[:, :, ::dilation]
    center_idx = jnp.broadcast_to(
        jnp.arange(n, dtype=nn_idx.dtype)[None, :, None], (b, n, k))
    return jnp.stack((nn_idx, center_idx), axis=0)


def _index_sel(x, idx):
    x_sq = jnp.squeeze(x, -1)
    return jax.vmap(lambda xb, ib: xb[:, ib])(x_sq, idx)


def _conv_bn_relu(x, W, bb, gamma, beta):
    y = jnp.einsum('oc,bcnk->bonk', W, x) + bb[None, :, None, None]
    mean = jnp.mean(y, axis=(0, 2, 3), keepdims=True)
    var = jnp.var(y, axis=(0, 2, 3), keepdims=True)
    y = (y - mean) / jnp.sqrt(var + 1e-5)
    y = y * gamma[None, :, None, None] + beta[None, :, None, None]
    return jax.nn.relu(y)


def _edge_conv(x, edge_index, W, bb, gamma, beta):
    x_i = _index_sel(x, edge_index[1])
    x_j = _index_sel(x, edge_index[0])
    out = _conv_bn_relu(jnp.concatenate([x_i, x_j - x_i], axis=1), W, bb, gamma, beta)
    return jnp.max(out, axis=-1, keepdims=True)


def kernel(inputs, W_head, b_head, g_head, be_head, W_blocks, b_blocks, g_blocks, be_blocks):
    topo_list = []
    topo = _knn_graph(jax.lax.stop_gradient(inputs[:, 0:3]), KK, 1)
    topo_list.append(topo[0])
    feat = _edge_conv(inputs, topo, W_head, b_head, g_head, be_head)
    for i in range(NBLK - 1):
        edge_index = _knn_graph(jax.lax.stop_gradient(feat), KK, 1 + i)
        out = _edge_conv(feat, edge_index, W_blocks[i], b_blocks[i], g_blocks[i], be_blocks[i])
        feat = out + feat
        topo_list.append(edge_index[0])
    out_feat = jnp.swapaxes(jnp.squeeze(feat, -1), 1, 2)
    return (out_feat, jnp.stack(topo_list, axis=0))


# speedup vs baseline: 1.6663x; 1.6649x over previous
import functools

import jax
import jax.numpy as jnp
from jax.experimental import pallas as pl
from jax.experimental.pallas import tpu as pltpu

KK = 20
NBLK = 7
NFILT = 16
CIN = 4
BB = 4
NN = 4096

ROW_BLK = 512
OUT_W = 32


def _knn_body(dil, n_extract, xt_blk_ref, xt_all_ref, sq_blk_ref, sq_all_ref,
              out_ref, dist_ref, idx_ref):
    a = xt_blk_ref[0]
    x = xt_all_ref[0]
    m = jax.lax.dot_general(a, x, (((0,), (0,)), ((), ())),
                            preferred_element_type=jnp.float32)
    x_inner = -2.0 * m
    sq_r = sq_blk_ref[0]
    sq_a = sq_all_ref[0]
    d = (jnp.transpose(sq_r) + x_inner) + sq_a
    dist_ref[:] = -d

    rb, n = dist_ref.shape
    iota = jax.lax.broadcasted_iota(jnp.int32, (rb, n), 1)
    col_iota = jax.lax.broadcasted_iota(jnp.int32, (rb, OUT_W), 1)
    idx_ref[:] = jnp.zeros((rb, OUT_W), jnp.int32)

    def step(t, _):
        dd = dist_ref[:]
        gmax = jnp.max(dd, axis=1, keepdims=True)
        pos = jnp.min(jnp.where(dd == gmax, iota, n), axis=1, keepdims=True)
        col = t // dil
        hit = jnp.logical_and(col_iota == col, t == col * dil)
        idx_ref[:] = jnp.where(hit, pos, idx_ref[:])
        dist_ref[:] = jnp.where(iota == pos, -jnp.inf, dd)
        return 0

    jax.lax.fori_loop(0, n_extract, step, 0)
    out_ref[0] = idx_ref[:]


def _knn_pallas(xt, dilation):
    b, n, c = xt.shape
    xt_t = jnp.swapaxes(xt, 1, 2)
    sq = jnp.sum(xt * xt, axis=-1)[:, None, :]
    n_extract = (KK - 1) * dilation + 1
    grid = (b, n // ROW_BLK)
    out = pl.pallas_call(
        functools.partial(_knn_body, dilation, n_extract),
        grid=grid,
        in_specs=[
            pl.BlockSpec((1, c, ROW_BLK), lambda i, j: (i, 0, j)),
            pl.BlockSpec((1, c, n), lambda i, j: (i, 0, 0)),
            pl.BlockSpec((1, 1, ROW_BLK), lambda i, j: (i, 0, j)),
            pl.BlockSpec((1, 1, n), lambda i, j: (i, 0, 0)),
        ],
        out_specs=pl.BlockSpec((1, ROW_BLK, OUT_W), lambda i, j: (i, j, 0)),
        out_shape=jax.ShapeDtypeStruct((b, n, OUT_W), jnp.int32),
        scratch_shapes=[
            pltpu.VMEM((ROW_BLK, n), jnp.float32),
            pltpu.VMEM((ROW_BLK, OUT_W), jnp.int32),
        ],
    )(xt_t, xt_t, sq, sq)
    return out[:, :, :KK]


def _knn_graph(x, k, dilation):
    xt = jnp.squeeze(x, -1).transpose(0, 2, 1)
    b, n, _ = xt.shape
    nn_idx = _knn_pallas(xt, dilation)
    center_idx = jnp.broadcast_to(
        jnp.arange(n, dtype=nn_idx.dtype)[None, :, None], (b, n, k))
    return jnp.stack((nn_idx, center_idx), axis=0)


def _index_sel(x, idx):
    x_sq = jnp.squeeze(x, -1)
    return jax.vmap(lambda xb, ib: xb[:, ib])(x_sq, idx)


def _conv_bn_relu(x, W, bb, gamma, beta):
    y = jnp.einsum('oc,bcnk->bonk', W, x) + bb[None, :, None, None]
    mean = jnp.mean(y, axis=(0, 2, 3), keepdims=True)
    var = jnp.var(y, axis=(0, 2, 3), keepdims=True)
    y = (y - mean) / jnp.sqrt(var + 1e-5)
    y = y * gamma[None, :, None, None] + beta[None, :, None, None]
    return jax.nn.relu(y)


def _edge_conv(x, edge_index, W, bb, gamma, beta):
    x_i = _index_sel(x, edge_index[1])
    x_j = _index_sel(x, edge_index[0])
    out = _conv_bn_relu(jnp.concatenate([x_i, x_j - x_i], axis=1), W, bb, gamma, beta)
    return jnp.max(out, axis=-1, keepdims=True)


def kernel(inputs, W_head, b_head, g_head, be_head, W_blocks, b_blocks, g_blocks, be_blocks):
    topo_list = []
    topo = _knn_graph(jax.lax.stop_gradient(inputs[:, 0:3]), KK, 1)
    topo_list.append(topo[0])
    feat = _edge_conv(inputs, topo, W_head, b_head, g_head, be_head)
    for i in range(NBLK - 1):
        edge_index = _knn_graph(jax.lax.stop_gradient(feat), KK, 1 + i)
        out = _edge_conv(feat, edge_index, W_blocks[i], b_blocks[i], g_blocks[i], be_blocks[i])
        feat = out + feat
        topo_list.append(edge_index[0])
    out_feat = jnp.swapaxes(jnp.squeeze(feat, -1), 1, 2)
    return (out_feat, jnp.stack(topo_list, axis=0))


# speedup vs baseline: 2.1102x; 1.2664x over previous
import functools

import jax
import jax.numpy as jnp
from jax import lax
from jax.experimental import pallas as pl
from jax.experimental.pallas import tpu as pltpu
from jax.experimental.pallas import tpu_sc as plsc

KK = 20
NBLK = 7
NFILT = 16
CIN = 4
BB = 4
NN = 4096

ROW_BLK = 512
OUT_W = 32
GRP = 4
NCHUNK = NN // 16
NEGINF = float("-inf")


def _dist_body(xt_blk_ref, xt_all_ref, sq_blk_ref, sq_all_ref, out_ref):
    a = xt_blk_ref[0]
    x = xt_all_ref[0]
    m = jax.lax.dot_general(a, x, (((0,), (0,)), ((), ())),
                            preferred_element_type=jnp.float32)
    x_inner = -2.0 * m
    sq_r = sq_blk_ref[0]
    sq_a = sq_all_ref[0]
    d = (jnp.transpose(sq_r) + x_inner) + sq_a
    out_ref[0] = -d


def _neg_adj_pallas(xt):
    b, n, c = xt.shape
    xt_t = jnp.swapaxes(xt, 1, 2)
    sq = jnp.sum(xt * xt, axis=-1)[:, None, :]
    grid = (b, n // ROW_BLK)
    return pl.pallas_call(
        _dist_body,
        grid=grid,
        in_specs=[
            pl.BlockSpec((1, c, ROW_BLK), lambda i, j: (i, 0, j)),
            pl.BlockSpec((1, c, n), lambda i, j: (i, 0, 0)),
            pl.BlockSpec((1, 1, ROW_BLK), lambda i, j: (i, 0, j)),
            pl.BlockSpec((1, 1, n), lambda i, j: (i, 0, 0)),
        ],
        out_specs=pl.BlockSpec((1, ROW_BLK, n), lambda i, j: (i, j, 0)),
        out_shape=jax.ShapeDtypeStruct((b, n, n), jnp.float32),
    )(xt_t, xt_t, sq, sq)


def _scal(v):
    if getattr(v, "ndim", 0) == 0:
        return v
    return jnp.max(v)


def _ffs(mask):
    return _scal(plsc.all_reduce_ffs(mask))


def _sc_topk_body(dil, n_extract, rows_per_worker, nc,
                  neg_hbm, out_hbm, row_ref, cm_ref, out_ref):
    wid = lax.axis_index("s") * nc + lax.axis_index("c")
    base = wid * rows_per_worker
    ngrp = rows_per_worker // GRP
    iota16 = lax.iota(jnp.int32, 16)
    addvecs = [iota16 * 16 + ((u + iota16) & 15) for u in range(16)]

    def group_body(gi, _):
        row0 = base + gi * GRP
        pltpu.sync_copy(neg_hbm.at[pl.ds(row0, GRP)], row_ref)

        def cm_build(g16, l2s):
            new = []
            for rr in range(GRP):
                cmv = None
                for u in range(16):
                    v = plsc.load_gather(
                        row_ref,
                        [jnp.full((16,), rr, jnp.int32),
                         g16 * 256 + addvecs[u]])
                    cmv = v if cmv is None else jnp.maximum(cmv, v)
                cm_ref[rr, pl.ds(16 * g16, 16)] = cmv
                new.append(jnp.where(iota16 == g16, _scal(jnp.max(cmv)),
                                     l2s[rr]))
            return tuple(new)

        l2s = lax.fori_loop(
            0, 16, cm_build,
            tuple(jnp.full((16,), NEGINF, jnp.float32) for _ in range(GRP)))
        zero = jnp.zeros((16,), jnp.int32)
        carries = [(l2s[rr], zero, zero) for rr in range(GRP)]

        def ext_step(t, carry):
            col = t // dil
            hit = t == col * dil
            new = []
            for rr in range(GRP):
                l2, a0, a1 = carry[rr]
                g = _scal(jnp.max(l2))
                hi = _ffs(l2 == g)
                cmv = cm_ref[rr, pl.ds(hi * 16, 16)]
                lo = _ffs(cmv == g)
                c = hi * 16 + lo
                ch = row_ref[rr, pl.ds(c * 16, 16)]
                lpos = _ffs(ch == g)
                pos = c * 16 + lpos
                ch2 = jnp.where(iota16 == lpos, NEGINF, ch)
                row_ref[rr, pl.ds(c * 16, 16)] = ch2
                cmv2 = jnp.where(iota16 == lo, _scal(jnp.max(ch2)), cmv)
                cm_ref[rr, pl.ds(hi * 16, 16)] = cmv2
                l2n = jnp.where(iota16 == hi, _scal(jnp.max(cmv2)), l2)
                a0n = jnp.where(jnp.logical_and(hit, iota16 == col), pos, a0)
                a1n = jnp.where(jnp.logical_and(hit, iota16 == col - 16), pos, a1)
                new.append((l2n, a0n, a1n))
            return tuple(new)

        res = lax.fori_loop(0, n_extract, ext_step, tuple(carries))
        for rr in range(GRP):
            _, a0, a1 = res[rr]
            out_ref[rr, pl.ds(0, 16)] = a0
            out_ref[rr, pl.ds(16, 16)] = a1
        pltpu.sync_copy(out_ref, out_hbm.at[pl.ds(row0, GRP)])
        return 0

    lax.fori_loop(0, ngrp, group_body, 0)


def _sc_topk(neg2d, dil):
    rows, n = neg2d.shape
    nc, ns = 2, 16
    nw = nc * ns
    rpw = rows // nw
    n_extract = (KK - 1) * dil + 1
    mesh = plsc.VectorSubcoreMesh(core_axis_name="c", subcore_axis_name="s",
                                  num_cores=nc, num_subcores=ns)
    k = pl.kernel(
        functools.partial(_sc_topk_body, dil, n_extract, rpw, nc),
        mesh=mesh,
        out_type=jax.ShapeDtypeStruct((rows, OUT_W), jnp.int32),
        scratch_types=[
            pltpu.VMEM((GRP, n), jnp.float32),
            pltpu.VMEM((GRP, NCHUNK), jnp.float32),
            pltpu.VMEM((GRP, OUT_W), jnp.int32),
        ],
        compiler_params=pltpu.CompilerParams(needs_layout_passes=False),
    )
    return k(neg2d)


def _knn_pallas(xt, dilation):
    b, n, c = xt.shape
    neg = _neg_adj_pallas(xt)
    idx = _sc_topk(neg.reshape(b * n, n), dilation)
    return idx.reshape(b, n, OUT_W)[:, :, :KK]


def _knn_graph(x, k, dilation):
    xt = jnp.squeeze(x, -1).transpose(0, 2, 1)
    b, n, _ = xt.shape
    nn_idx = _knn_pallas(xt, dilation)
    center_idx = jnp.broadcast_to(
        jnp.arange(n, dtype=nn_idx.dtype)[None, :, None], (b, n, k))
    return jnp.stack((nn_idx, center_idx), axis=0)


def _index_sel(x, idx):
    x_sq = jnp.squeeze(x, -1)
    return jax.vmap(lambda xb, ib: xb[:, ib])(x_sq, idx)


def _conv_bn_relu(x, W, bb, gamma, beta):
    y = jnp.einsum('oc,bcnk->bonk', W, x) + bb[None, :, None, None]
    mean = jnp.mean(y, axis=(0, 2, 3), keepdims=True)
    var = jnp.var(y, axis=(0, 2, 3), keepdims=True)
    y = (y - mean) / jnp.sqrt(var + 1e-5)
    y = y * gamma[None, :, None, None] + beta[None, :, None, None]
    return jax.nn.relu(y)


def _edge_conv(x, edge_index, W, bb, gamma, beta):
    x_i = _index_sel(x, edge_index[1])
    x_j = _index_sel(x, edge_index[0])
    out = _conv_bn_relu(jnp.concatenate([x_i, x_j - x_i], axis=1), W, bb, gamma, beta)
    return jnp.max(out, axis=-1, keepdims=True)


def kernel(inputs, W_head, b_head, g_head, be_head, W_blocks, b_blocks, g_blocks, be_blocks):
    topo_list = []
    topo = _knn_graph(jax.lax.stop_gradient(inputs[:, 0:3]), KK, 1)
    topo_list.append(topo[0])
    feat = _edge_conv(inputs, topo, W_head, b_head, g_head, be_head)
    for i in range(NBLK - 1):
        edge_index = _knn_graph(jax.lax.stop_gradient(feat), KK, 1 + i)
        out = _edge_conv(feat, edge_index, W_blocks[i], b_blocks[i], g_blocks[i], be_blocks[i])
        feat = out + feat
        topo_list.append(edge_index[0])
    out_feat = jnp.swapaxes(jnp.squeeze(feat, -1), 1, 2)
    return (out_feat, jnp.stack(topo_list, axis=0))


# speedup vs baseline: 8.7753x; 4.1586x over previous
import functools

import jax
import jax.numpy as jnp
from jax import lax
from jax.experimental import pallas as pl
from jax.experimental.pallas import tpu as pltpu
from jax.experimental.pallas import tpu_sc as plsc

KK = 20
NBLK = 7
NFILT = 16
CIN = 4
BB = 4
NN = 4096

ROW_BLK = 512
OUT_W = 32
GRP = 4
NCHUNK = NN // 16
NEGINF = float("-inf")


def _dist_body(xt_blk_ref, xt_all_ref, sq_blk_ref, sq_all_ref, out_ref):
    a = xt_blk_ref[0]
    x = xt_all_ref[0]
    m = jax.lax.dot_general(a, x, (((0,), (0,)), ((), ())),
                            preferred_element_type=jnp.float32)
    x_inner = -2.0 * m
    sq_r = sq_blk_ref[0]
    sq_a = sq_all_ref[0]
    d = (jnp.transpose(sq_r) + x_inner) + sq_a
    out_ref[0] = -d


def _neg_adj_pallas(xt):
    b, n, c = xt.shape
    xt_t = jnp.swapaxes(xt, 1, 2)
    sq = jnp.sum(xt * xt, axis=-1)[:, None, :]
    grid = (b, n // ROW_BLK)
    return pl.pallas_call(
        _dist_body,
        grid=grid,
        in_specs=[
            pl.BlockSpec((1, c, ROW_BLK), lambda i, j: (i, 0, j)),
            pl.BlockSpec((1, c, n), lambda i, j: (i, 0, 0)),
            pl.BlockSpec((1, 1, ROW_BLK), lambda i, j: (i, 0, j)),
            pl.BlockSpec((1, 1, n), lambda i, j: (i, 0, 0)),
        ],
        out_specs=pl.BlockSpec((1, ROW_BLK, n), lambda i, j: (i, j, 0)),
        out_shape=jax.ShapeDtypeStruct((b, n, n), jnp.float32),
    )(xt_t, xt_t, sq, sq)


def _scal(v):
    if getattr(v, "ndim", 0) == 0:
        return v
    return jnp.max(v)


def _ffs(mask):
    return _scal(plsc.all_reduce_ffs(mask))


def _sc_topk_body(dil, n_extract, rows_per_worker, nc,
                  neg_hbm, out_hbm, row_ref, cm_ref, out_ref):
    wid = lax.axis_index("s") * nc + lax.axis_index("c")
    base = wid * rows_per_worker
    ngrp = rows_per_worker // GRP
    iota16 = lax.iota(jnp.int32, 16)
    addvecs = [iota16 * 16 + ((u + iota16) & 15) for u in range(16)]

    def group_body(gi, _):
        row0 = base + gi * GRP
        pltpu.sync_copy(neg_hbm.at[pl.ds(row0, GRP)], row_ref)

        def cm_build(g16, l2s):
            new = []
            for rr in range(GRP):
                cmv = None
                for u in range(16):
                    v = plsc.load_gather(
                        row_ref,
                        [jnp.full((16,), rr, jnp.int32),
                         g16 * 256 + addvecs[u]])
                    cmv = v if cmv is None else jnp.maximum(cmv, v)
                cm_ref[rr, pl.ds(16 * g16, 16)] = cmv
                new.append(jnp.where(iota16 == g16, _scal(jnp.max(cmv)),
                                     l2s[rr]))
            return tuple(new)

        l2s = lax.fori_loop(
            0, 16, cm_build,
            tuple(jnp.full((16,), NEGINF, jnp.float32) for _ in range(GRP)))
        zero = jnp.zeros((16,), jnp.int32)
        carries = [(l2s[rr], zero, zero) for rr in range(GRP)]

        def ext_step(t, carry):
            col = t // dil
            hit = t == col * dil
            new = []
            for rr in range(GRP):
                l2, a0, a1 = carry[rr]
                g = _scal(jnp.max(l2))
                hi = _ffs(l2 == g)
                cmv = cm_ref[rr, pl.ds(hi * 16, 16)]
                lo = _ffs(cmv == g)
                c = hi * 16 + lo
                ch = row_ref[rr, pl.ds(c * 16, 16)]
                lpos = _ffs(ch == g)
                pos = c * 16 + lpos
                ch2 = jnp.where(iota16 == lpos, NEGINF, ch)
                row_ref[rr, pl.ds(c * 16, 16)] = ch2
                cmv2 = jnp.where(iota16 == lo, _scal(jnp.max(ch2)), cmv)
                cm_ref[rr, pl.ds(hi * 16, 16)] = cmv2
                l2n = jnp.where(iota16 == hi, _scal(jnp.max(cmv2)), l2)
                a0n = jnp.where(jnp.logical_and(hit, iota16 == col), pos, a0)
                a1n = jnp.where(jnp.logical_and(hit, iota16 == col - 16), pos, a1)
                new.append((l2n, a0n, a1n))
            return tuple(new)

        res = lax.fori_loop(0, n_extract, ext_step, tuple(carries))
        for rr in range(GRP):
            _, a0, a1 = res[rr]
            out_ref[rr, pl.ds(0, 16)] = a0
            out_ref[rr, pl.ds(16, 16)] = a1
        pltpu.sync_copy(out_ref, out_hbm.at[pl.ds(row0, GRP)])
        return 0

    lax.fori_loop(0, ngrp, group_body, 0)


def _sc_topk(neg2d, dil):
    rows, n = neg2d.shape
    nc, ns = 2, 16
    nw = nc * ns
    rpw = rows // nw
    n_extract = (KK - 1) * dil + 1
    mesh = plsc.VectorSubcoreMesh(core_axis_name="c", subcore_axis_name="s",
                                  num_cores=nc, num_subcores=ns)
    k = pl.kernel(
        functools.partial(_sc_topk_body, dil, n_extract, rpw, nc),
        mesh=mesh,
        out_type=jax.ShapeDtypeStruct((rows, OUT_W), jnp.int32),
        scratch_types=[
            pltpu.VMEM((GRP, n), jnp.float32),
            pltpu.VMEM((GRP, NCHUNK), jnp.float32),
            pltpu.VMEM((GRP, OUT_W), jnp.int32),
        ],
        compiler_params=pltpu.CompilerParams(needs_layout_passes=False),
    )
    return k(neg2d)


def _knn_pallas(xt, dilation):
    b, n, c = xt.shape
    neg = _neg_adj_pallas(xt)
    idx = _sc_topk(neg.reshape(b * n, n), dilation)
    return idx.reshape(b, n, OUT_W)[:, :, :KK]


def _knn_graph(x, k, dilation):
    xt = jnp.squeeze(x, -1).transpose(0, 2, 1)
    return _knn_pallas(xt, dilation)


GCH = 2048


def _sc_gather_body(epw, nch, table_hbm, idx_hbm, out_hbm, idx_v, rows_v, sem):
    wid = lax.axis_index("s") * 2 + lax.axis_index("c")
    base = wid * epw

    def chunk(ci, _):
        off = base + ci * GCH
        pltpu.sync_copy(idx_hbm.at[pl.ds(off, GCH)], idx_v)
        pltpu.async_copy(table_hbm.at[idx_v], rows_v, sem).wait()
        pltpu.sync_copy(rows_v, out_hbm.at[pl.ds(off, GCH)])
        return 0

    lax.fori_loop(0, nch, chunk, 0)


def _sc_gather(table, idx_flat):
    e = idx_flat.shape[0]
    nc, ns = 2, 16
    nw = nc * ns
    epw = e // nw
    mesh = plsc.VectorSubcoreMesh(core_axis_name="c", subcore_axis_name="s",
                                  num_cores=nc, num_subcores=ns)
    k = pl.kernel(
        functools.partial(_sc_gather_body, epw, epw // GCH),
        mesh=mesh,
        out_type=jax.ShapeDtypeStruct((e, 16), jnp.float32),
        scratch_types=[
            pltpu.VMEM((GCH,), jnp.int32),
            pltpu.VMEM((GCH, 16), jnp.float32),
            pltpu.SemaphoreType.DMA,
        ],
        compiler_params=pltpu.CompilerParams(use_tc_tiling_on_sc=False),
    )
    return k(table, idx_flat)


def _gather_xj(x_sq, nn_idx):
    b, c, n = x_sq.shape
    k = nn_idx.shape[-1]
    table = jnp.swapaxes(x_sq, 1, 2).reshape(b * n, c)
    if c < 16:
        table = jnp.pad(table, ((0, 0), (0, 16 - c)))
    idx_flat = (jnp.arange(b, dtype=jnp.int32)[:, None, None] * n
                + nn_idx).reshape(-1)
    rows = _sc_gather(table, idx_flat)
    return rows.reshape(b, n, k, 16)[:, :, :, :c].transpose(0, 3, 1, 2)


def _conv_bn_relu(x, W, bb, gamma, beta):
    y = jnp.einsum('oc,bcnk->bonk', W, x) + bb[None, :, None, None]
    mean = jnp.mean(y, axis=(0, 2, 3), keepdims=True)
    var = jnp.var(y, axis=(0, 2, 3), keepdims=True)
    y = (y - mean) / jnp.sqrt(var + 1e-5)
    y = y * gamma[None, :, None, None] + beta[None, :, None, None]
    return jax.nn.relu(y)


def _edge_conv(x, nn_idx, W, bb, gamma, beta):
    x_sq = jnp.squeeze(x, -1)
    k = nn_idx.shape[-1]
    x_i = jnp.broadcast_to(x_sq[:, :, :, None], x_sq.shape + (k,))
    x_j = _gather_xj(x_sq, nn_idx)
    out = _conv_bn_relu(jnp.concatenate([x_i, x_j - x_i], axis=1), W, bb, gamma, beta)
    return jnp.max(out, axis=-1, keepdims=True)


def kernel(inputs, W_head, b_head, g_head, be_head, W_blocks, b_blocks, g_blocks, be_blocks):
    topo_list = []
    topo = _knn_graph(jax.lax.stop_gradient(inputs[:, 0:3]), KK, 1)
    topo_list.append(topo)
    feat = _edge_conv(inputs, topo, W_head, b_head, g_head, be_head)
    for i in range(NBLK - 1):
        nn_idx = _knn_graph(jax.lax.stop_gradient(feat), KK, 1 + i)
        out = _edge_conv(feat, nn_idx, W_blocks[i], b_blocks[i], g_blocks[i], be_blocks[i])
        feat = out + feat
        topo_list.append(nn_idx)
    out_feat = jnp.swapaxes(jnp.squeeze(feat, -1), 1, 2)
    return (out_feat, jnp.stack(topo_list, axis=0))
